# baseline (speedup 1.0000x reference)
import functools

import jax
import jax.numpy as jnp
from jax import lax
from jax.experimental import pallas as pl
from jax.experimental.pallas import tpu as pltpu
from jax.experimental.pallas import tpu_sc as plsc

N_NODES = 10000
N_EDGES = 320000
D_FEAT = 128
N_CLASS = 40

NUM_CORES = 2
NUM_SUBCORES = 16
NUM_TILES = NUM_CORES * NUM_SUBCORES
EDGES_PER_TILE = N_EDGES // NUM_TILES
CHUNK = 128
NCHUNK = 80
PAD_EDGES = NCHUNK * CHUNK
GROUP = 16
NGROUP = NCHUNK // GROUP
ACC_ROWS = 10016
JUNK_ROW = N_NODES
ROWS_MAIN = 632
ROWS_TAIL = N_NODES - 15 * ROWS_MAIN


def _sc_aggregate_body(src_hbm, dst_hbm, feat_hbm, zeros_hbm, out_hbm,
                       sidx_v, didx_v, rows_v, acc_s, sem_a, sem_b):
  cid = lax.axis_index("c")
  sid = lax.axis_index("s")
  wid = cid * NUM_SUBCORES + sid

  pltpu.sync_copy(src_hbm.at[wid, pl.ds(0, GROUP)], sidx_v.at[0])
  pltpu.sync_copy(dst_hbm.at[wid, pl.ds(0, GROUP)], didx_v.at[0])
  pltpu.sync_copy(src_hbm.at[wid, pl.ds(GROUP, GROUP)], sidx_v.at[1])
  pltpu.sync_copy(dst_hbm.at[wid, pl.ds(GROUP, GROUP)], didx_v.at[1])

  row0 = sid * ROWS_MAIN

  def _init(nrows):
    @pl.when(cid == 0)
    def _():
      pltpu.sync_copy(feat_hbm.at[pl.ds(row0, nrows)],
                      acc_s.at[pl.ds(row0, nrows)])

    @pl.when(cid == 1)
    def _():
      pltpu.sync_copy(zeros_hbm.at[pl.ds(row0, nrows)],
                      acc_s.at[pl.ds(row0, nrows)])

  @pl.when(sid < NUM_SUBCORES - 1)
  def _():
    _init(ROWS_MAIN)

  @pl.when(sid == NUM_SUBCORES - 1)
  def _():
    _init(ROWS_TAIL)

  plsc.subcore_barrier()

  def _sidx(c):
    g = c // GROUP
    return sidx_v.at[lax.rem(g, 2), lax.rem(c, GROUP)]

  def _didx(c):
    g = c // GROUP
    return didx_v.at[lax.rem(g, 2), lax.rem(c, GROUP)]

  def _gather(c, b, sem):
    pltpu.async_copy(feat_hbm.at[_sidx(c)], rows_v.at[b], sem)

  def _wait(c, b, sem):
    pltpu.make_async_copy(feat_hbm.at[_sidx(c)], rows_v.at[b], sem).wait()

  def _scatter(c, b):
    pltpu.sync_copy(rows_v.at[b], acc_s.at[_didx(c)], add=True)

  def body(i, carry):
    c = 2 * i
    pltpu.sync_copy(feat_hbm.at[_sidx(c)], rows_v.at[0])
    pltpu.sync_copy(feat_hbm.at[_sidx(c + 1)], rows_v.at[1])

    g = c // GROUP

    @pl.when((lax.rem(c, GROUP) == GROUP - 2) & (g + 2 < NGROUP))
    def _():
      q = g + 2
      slot = lax.rem(q, 2)
      pltpu.sync_copy(src_hbm.at[wid, pl.ds(q * GROUP, GROUP)],
                      sidx_v.at[slot])
      pltpu.sync_copy(dst_hbm.at[wid, pl.ds(q * GROUP, GROUP)],
                      didx_v.at[slot])

    return carry

  lax.fori_loop(0, NCHUNK // 2, body, 0, unroll=False)

  plsc.subcore_barrier()

  @pl.when(sid < NUM_SUBCORES - 1)
  def _():
    pltpu.sync_copy(acc_s.at[pl.ds(row0, ROWS_MAIN)],
                    out_hbm.at[cid, pl.ds(row0, ROWS_MAIN)])

  @pl.when(sid == NUM_SUBCORES - 1)
  def _():
    pltpu.sync_copy(acc_s.at[pl.ds(row0, ROWS_TAIL)],
                    out_hbm.at[cid, pl.ds(row0, ROWS_TAIL)])


_sc_aggregate = functools.partial(
    pl.kernel,
    out_type=jax.ShapeDtypeStruct((NUM_CORES, N_NODES, D_FEAT), jnp.float32),
    mesh=plsc.VectorSubcoreMesh(core_axis_name="c", subcore_axis_name="s"),
    scratch_types=[
        pltpu.VMEM((2, GROUP, CHUNK), jnp.int32),
        pltpu.VMEM((2, GROUP, CHUNK), jnp.int32),
        pltpu.VMEM((2, CHUNK, D_FEAT), jnp.float32),
        pltpu.VMEM_SHARED((ACC_ROWS, D_FEAT), jnp.float32),
        pltpu.SemaphoreType.DMA,
        pltpu.SemaphoreType.DMA,
    ],
)(_sc_aggregate_body)


ROW_BLK = 2000


def _mlp1_body(p0_ref, p1_ref, w_ref, b_ref, out_ref):
  a = p0_ref[...] + p1_ref[...]
  h = jnp.dot(a, w_ref[...], preferred_element_type=jnp.float32) + b_ref[...]
  out_ref[...] = jnp.maximum(h, 0.0)


def _mlp2_body(p0_ref, p1_ref, w2_ref, b2_ref, w3_ref, b3_ref, out_ref):
  a = p0_ref[...] + p1_ref[...]
  h2 = jnp.dot(a, w2_ref[...], preferred_element_type=jnp.float32)
  h2 = jnp.maximum(h2 + b2_ref[...], 0.0)
  logits = jnp.dot(h2, w3_ref[...], preferred_element_type=jnp.float32)
  logits = logits + b3_ref[...]
  m = jnp.max(logits, axis=1, keepdims=True)
  lse = m + jnp.log(jnp.sum(jnp.exp(logits - m), axis=1, keepdims=True))
  out_ref[...] = logits - lse


def _row_block(i):
  return (i, 0)


def _full_block(i):
  return (0, 0)


_mlp1 = pl.pallas_call(
    _mlp1_body,
    grid=(N_NODES // ROW_BLK,),
    in_specs=[
        pl.BlockSpec((ROW_BLK, D_FEAT), _row_block),
        pl.BlockSpec((ROW_BLK, D_FEAT), _row_block),
        pl.BlockSpec((D_FEAT, D_FEAT), _full_block),
        pl.BlockSpec((1, D_FEAT), _full_block),
    ],
    out_specs=pl.BlockSpec((ROW_BLK, D_FEAT), _row_block),
    out_shape=jax.ShapeDtypeStruct((N_NODES, D_FEAT), jnp.float32),
)

_mlp2 = pl.pallas_call(
    _mlp2_body,
    grid=(N_NODES // ROW_BLK,),
    in_specs=[
        pl.BlockSpec((ROW_BLK, D_FEAT), _row_block),
        pl.BlockSpec((ROW_BLK, D_FEAT), _row_block),
        pl.BlockSpec((D_FEAT, D_FEAT), _full_block),
        pl.BlockSpec((1, D_FEAT), _full_block),
        pl.BlockSpec((D_FEAT, N_CLASS), _full_block),
        pl.BlockSpec((1, N_CLASS), _full_block),
    ],
    out_specs=pl.BlockSpec((ROW_BLK, N_CLASS), _row_block),
    out_shape=jax.ShapeDtypeStruct((N_NODES, N_CLASS), jnp.float32),
)


def _chunk_indices(idx, pad_value):
  per_tile = idx.reshape(NUM_TILES, EDGES_PER_TILE)
  padded = jnp.pad(per_tile, ((0, 0), (0, PAD_EDGES - EDGES_PER_TILE)),
                   constant_values=pad_value)
  return padded.reshape(NUM_TILES, NCHUNK, CHUNK)


@jax.jit
def kernel(x, edge_index, W1, b1, W2, b2, W3, b3):
  src = _chunk_indices(edge_index[0].astype(jnp.int32), 0)
  dst = _chunk_indices(edge_index[1].astype(jnp.int32), JUNK_ROW)
  zeros = jnp.zeros((N_NODES, D_FEAT), jnp.float32)

  p = _sc_aggregate(src, dst, x, zeros)
  h1 = _mlp1(p[0], p[1], W1, b1.reshape(1, D_FEAT))
  p2 = _sc_aggregate(src, dst, h1, zeros)
  return _mlp2(p2[0], p2[1], W2, b2.reshape(1, D_FEAT),
               W3, b3.reshape(1, N_CLASS))


# speedup vs baseline: 1.0075x; 1.0075x over previous
import functools

import jax
import jax.numpy as jnp
from jax import lax
from jax.experimental import pallas as pl
from jax.experimental.pallas import tpu as pltpu
from jax.experimental.pallas import tpu_sc as plsc

N_NODES = 10000
N_EDGES = 320000
D_FEAT = 128
N_CLASS = 40

NUM_CORES = 2
NUM_SUBCORES = 16
NUM_TILES = NUM_CORES * NUM_SUBCORES
EDGES_PER_TILE = N_EDGES // NUM_TILES
CHUNK = 128
NCHUNK = 80
PAD_EDGES = NCHUNK * CHUNK
GROUP = 16
NGROUP = NCHUNK // GROUP
ACC_ROWS = 10016
JUNK_ROW = N_NODES
ROWS_MAIN = 632
ROWS_TAIL = N_NODES - 15 * ROWS_MAIN


def _sc_aggregate_body(src_hbm, dst_hbm, feat_hbm, zeros_hbm, out_hbm,
                       sidx_v, didx_v, rows_v, acc_s, sem_a, sem_b):
  cid = lax.axis_index("c")
  sid = lax.axis_index("s")
  wid = cid * NUM_SUBCORES + sid

  pltpu.sync_copy(src_hbm.at[wid], sidx_v)
  pltpu.sync_copy(dst_hbm.at[wid], didx_v)

  row0 = sid * ROWS_MAIN

  def _init(nrows):
    @pl.when(cid == 0)
    def _():
      pltpu.sync_copy(feat_hbm.at[pl.ds(row0, nrows)],
                      acc_s.at[pl.ds(row0, nrows)])

    @pl.when(cid == 1)
    def _():
      pltpu.sync_copy(zeros_hbm.at[pl.ds(row0, nrows)],
                      acc_s.at[pl.ds(row0, nrows)])

  @pl.when(sid < NUM_SUBCORES - 1)
  def _():
    _init(ROWS_MAIN)

  @pl.when(sid == NUM_SUBCORES - 1)
  def _():
    _init(ROWS_TAIL)

  plsc.subcore_barrier()

  def body(j, carry):
    pltpu.sync_copy(feat_hbm.at[sidx_v.at[j]], rows_v)
    return carry

  lax.fori_loop(0, NCHUNK, body, 0, unroll=False)

  plsc.subcore_barrier()

  @pl.when(sid < NUM_SUBCORES - 1)
  def _():
    pltpu.sync_copy(acc_s.at[pl.ds(row0, ROWS_MAIN)],
                    out_hbm.at[cid, pl.ds(row0, ROWS_MAIN)])

  @pl.when(sid == NUM_SUBCORES - 1)
  def _():
    pltpu.sync_copy(acc_s.at[pl.ds(row0, ROWS_TAIL)],
                    out_hbm.at[cid, pl.ds(row0, ROWS_TAIL)])


_sc_aggregate = functools.partial(
    pl.kernel,
    out_type=jax.ShapeDtypeStruct((NUM_CORES, N_NODES, D_FEAT), jnp.float32),
    mesh=plsc.VectorSubcoreMesh(core_axis_name="c", subcore_axis_name="s"),
    scratch_types=[
        pltpu.VMEM((NCHUNK, CHUNK), jnp.int32),
        pltpu.VMEM((NCHUNK, CHUNK), jnp.int32),
        pltpu.VMEM((CHUNK, D_FEAT), jnp.float32),
        pltpu.VMEM_SHARED((ACC_ROWS, D_FEAT), jnp.float32),
        pltpu.SemaphoreType.DMA,
        pltpu.SemaphoreType.DMA,
    ],
)(_sc_aggregate_body)


ROW_BLK = 2000


def _mlp1_body(p0_ref, p1_ref, w_ref, b_ref, out_ref):
  a = p0_ref[...] + p1_ref[...]
  h = jnp.dot(a, w_ref[...], preferred_element_type=jnp.float32) + b_ref[...]
  out_ref[...] = jnp.maximum(h, 0.0)


def _mlp2_body(p0_ref, p1_ref, w2_ref, b2_ref, w3_ref, b3_ref, out_ref):
  a = p0_ref[...] + p1_ref[...]
  h2 = jnp.dot(a, w2_ref[...], preferred_element_type=jnp.float32)
  h2 = jnp.maximum(h2 + b2_ref[...], 0.0)
  logits = jnp.dot(h2, w3_ref[...], preferred_element_type=jnp.float32)
  logits = logits + b3_ref[...]
  m = jnp.max(logits, axis=1, keepdims=True)
  lse = m + jnp.log(jnp.sum(jnp.exp(logits - m), axis=1, keepdims=True))
  out_ref[...] = logits - lse


def _row_block(i):
  return (i, 0)


def _full_block(i):
  return (0, 0)


_mlp1 = pl.pallas_call(
    _mlp1_body,
    grid=(N_NODES // ROW_BLK,),
    in_specs=[
        pl.BlockSpec((ROW_BLK, D_FEAT), _row_block),
        pl.BlockSpec((ROW_BLK, D_FEAT), _row_block),
        pl.BlockSpec((D_FEAT, D_FEAT), _full_block),
        pl.BlockSpec((1, D_FEAT), _full_block),
    ],
    out_specs=pl.BlockSpec((ROW_BLK, D_FEAT), _row_block),
    out_shape=jax.ShapeDtypeStruct((N_NODES, D_FEAT), jnp.float32),
)

_mlp2 = pl.pallas_call(
    _mlp2_body,
    grid=(N_NODES // ROW_BLK,),
    in_specs=[
        pl.BlockSpec((ROW_BLK, D_FEAT), _row_block),
        pl.BlockSpec((ROW_BLK, D_FEAT), _row_block),
        pl.BlockSpec((D_FEAT, D_FEAT), _full_block),
        pl.BlockSpec((1, D_FEAT), _full_block),
        pl.BlockSpec((D_FEAT, N_CLASS), _full_block),
        pl.BlockSpec((1, N_CLASS), _full_block),
    ],
    out_specs=pl.BlockSpec((ROW_BLK, N_CLASS), _row_block),
    out_shape=jax.ShapeDtypeStruct((N_NODES, N_CLASS), jnp.float32),
)


def _chunk_indices(idx, pad_value):
  per_tile = idx.reshape(NUM_TILES, EDGES_PER_TILE)
  padded = jnp.pad(per_tile, ((0, 0), (0, PAD_EDGES - EDGES_PER_TILE)),
                   constant_values=pad_value)
  return padded.reshape(NUM_TILES, NCHUNK, CHUNK)


@jax.jit
def kernel(x, edge_index, W1, b1, W2, b2, W3, b3):
  src = _chunk_indices(edge_index[0].astype(jnp.int32), 0)
  dst = _chunk_indices(edge_index[1].astype(jnp.int32), JUNK_ROW)
  zeros = jnp.zeros((N_NODES, D_FEAT), jnp.float32)

  p = _sc_aggregate(src, dst, x, zeros)
  h1 = _mlp1(p[0], p[1], W1, b1.reshape(1, D_FEAT))
  p2 = _sc_aggregate(src, dst, h1, zeros)
  return _mlp2(p2[0], p2[1], W2, b2.reshape(1, D_FEAT),
               W3, b3.reshape(1, N_CLASS))


# speedup vs baseline: 1.5222x; 1.5109x over previous
import functools

import jax
import jax.numpy as jnp
from jax import lax
from jax.experimental import pallas as pl
from jax.experimental.pallas import tpu as pltpu
from jax.experimental.pallas import tpu_sc as plsc

N_NODES = 10000
N_EDGES = 320000
D_FEAT = 128
HALF = D_FEAT // 2
N_CLASS = 40

NUM_CORES = 2
NUM_SUBCORES = 16
EDGES_PER_TILE = N_EDGES // NUM_SUBCORES
CHUNK = 128
NCHUNK = 160
PAD_EDGES = NCHUNK * CHUNK
NBUF = 8
GROUP = 16
NGROUP = NCHUNK // GROUP
ACC_ROWS = 10016
JUNK_ROW = N_NODES
ROWS_MAIN = 632
ROWS_TAIL = N_NODES - 15 * ROWS_MAIN


def _sc_aggregate_body(src_hbm, dst_hbm, feat_hbm, out_hbm,
                       sidx_v, didx_v, rows_v, acc_s, *sems):
  cid = lax.axis_index("c")
  sid = lax.axis_index("s")

  def _refill(q):
    slot = lax.rem(q, 2)
    pltpu.sync_copy(src_hbm.at[cid, sid, pl.ds(q * GROUP, GROUP)],
                    sidx_v.at[slot])
    pltpu.sync_copy(dst_hbm.at[cid, sid, pl.ds(q * GROUP, GROUP)],
                    didx_v.at[slot])

  _refill(0)
  _refill(1)

  row0 = sid * ROWS_MAIN
  feat0 = cid * N_NODES + row0

  def _init(nrows):
    pltpu.sync_copy(feat_hbm.at[pl.ds(feat0, nrows)],
                    acc_s.at[pl.ds(row0, nrows)])

  @pl.when(sid < NUM_SUBCORES - 1)
  def _():
    _init(ROWS_MAIN)

  @pl.when(sid == NUM_SUBCORES - 1)
  def _():
    _init(ROWS_TAIL)

  plsc.subcore_barrier()

  def _sidx(c):
    return sidx_v.at[lax.rem(c // GROUP, 2), lax.rem(c, GROUP)]

  def _didx(c):
    return didx_v.at[lax.rem(c // GROUP, 2), lax.rem(c, GROUP)]

  def _gather(c, b):
    pltpu.async_copy(feat_hbm.at[_sidx(c)], rows_v.at[b], sems[b])

  def _wait(c, b):
    pltpu.make_async_copy(feat_hbm.at[_sidx(c)], rows_v.at[b],
                          sems[b]).wait()

  def _scatter(c, b):
    pltpu.sync_copy(rows_v.at[b], acc_s.at[_didx(c)], add=True)

  for b in range(NBUF):
    _gather(b, b)

  def body(i, carry):
    c0 = NBUF * i
    for b in range(NBUF):
      c = c0 + b
      _wait(c, b)
      _scatter(c, b)

      @pl.when(c + NBUF < NCHUNK)
      def _():
        _gather(c + NBUF, b)

      g = c // GROUP

      @pl.when((lax.rem(c, GROUP) == GROUP - 1) & (g + 2 < NGROUP))
      def _():
        _refill(g + 2)

    return carry

  lax.fori_loop(0, NCHUNK // NBUF, body, 0, unroll=False)

  plsc.subcore_barrier()

  @pl.when(sid < NUM_SUBCORES - 1)
  def _():
    pltpu.sync_copy(acc_s.at[pl.ds(row0, ROWS_MAIN)],
                    out_hbm.at[cid, pl.ds(row0, ROWS_MAIN)])

  @pl.when(sid == NUM_SUBCORES - 1)
  def _():
    pltpu.sync_copy(acc_s.at[pl.ds(row0, ROWS_TAIL)],
                    out_hbm.at[cid, pl.ds(row0, ROWS_TAIL)])


_sc_aggregate = functools.partial(
    pl.kernel,
    out_type=jax.ShapeDtypeStruct((NUM_CORES, N_NODES, HALF), jnp.float32),
    mesh=plsc.VectorSubcoreMesh(core_axis_name="c", subcore_axis_name="s"),
    compiler_params=pltpu.CompilerParams(use_tc_tiling_on_sc=False),
    scratch_types=[
        pltpu.VMEM((2, GROUP, CHUNK), jnp.int32),
        pltpu.VMEM((2, GROUP, CHUNK), jnp.int32),
        pltpu.VMEM((NBUF, CHUNK, HALF), jnp.float32),
        pltpu.VMEM_SHARED((ACC_ROWS, HALF), jnp.float32),
    ] + [pltpu.SemaphoreType.DMA] * NBUF,
)(_sc_aggregate_body)


ROW_BLK = 2000


def _mlp1_body(p_ref, w_ref, b_ref, out_ref):
  a = jnp.concatenate([p_ref[0], p_ref[1]], axis=1)
  h = jnp.dot(a, w_ref[...], preferred_element_type=jnp.float32) + b_ref[...]
  h = jnp.maximum(h, 0.0)
  out_ref[0, :, :] = h[:, :HALF]
  out_ref[1, :, :] = h[:, HALF:]


def _mlp2_body(p_ref, w2_ref, b2_ref, w3_ref, b3_ref, out_ref):
  a = jnp.concatenate([p_ref[0], p_ref[1]], axis=1)
  h2 = jnp.dot(a, w2_ref[...], preferred_element_type=jnp.float32)
  h2 = jnp.maximum(h2 + b2_ref[...], 0.0)
  logits = jnp.dot(h2, w3_ref[...], preferred_element_type=jnp.float32)
  logits = logits + b3_ref[...]
  m = jnp.max(logits, axis=1, keepdims=True)
  lse = m + jnp.log(jnp.sum(jnp.exp(logits - m), axis=1, keepdims=True))
  out_ref[...] = logits - lse


def _split_block(i):
  return (0, i, 0)


def _row_block(i):
  return (i, 0)


def _full_block(i):
  return (0, 0)


_mlp1 = pl.pallas_call(
    _mlp1_body,
    grid=(N_NODES // ROW_BLK,),
    in_specs=[
        pl.BlockSpec((NUM_CORES, ROW_BLK, HALF), _split_block),
        pl.BlockSpec((D_FEAT, D_FEAT), _full_block),
        pl.BlockSpec((1, D_FEAT), _full_block),
    ],
    out_specs=pl.BlockSpec((NUM_CORES, ROW_BLK, HALF), _split_block),
    out_shape=jax.ShapeDtypeStruct((NUM_CORES, N_NODES, HALF), jnp.float32),
)

_mlp2 = pl.pallas_call(
    _mlp2_body,
    grid=(N_NODES // ROW_BLK,),
    in_specs=[
        pl.BlockSpec((NUM_CORES, ROW_BLK, HALF), _split_block),
        pl.BlockSpec((D_FEAT, D_FEAT), _full_block),
        pl.BlockSpec((1, D_FEAT), _full_block),
        pl.BlockSpec((D_FEAT, N_CLASS), _full_block),
        pl.BlockSpec((1, N_CLASS), _full_block),
    ],
    out_specs=pl.BlockSpec((ROW_BLK, N_CLASS), _row_block),
    out_shape=jax.ShapeDtypeStruct((N_NODES, N_CLASS), jnp.float32),
)


def _chunk_indices(idx, pad_value):
  per_tile = idx.reshape(NUM_SUBCORES, EDGES_PER_TILE)
  padded = jnp.pad(per_tile, ((0, 0), (0, PAD_EDGES - EDGES_PER_TILE)),
                   constant_values=pad_value)
  return padded.reshape(NUM_SUBCORES, NCHUNK, CHUNK)


@jax.jit
def kernel(x, edge_index, W1, b1, W2, b2, W3, b3):
  src = _chunk_indices(edge_index[0].astype(jnp.int32), 0)
  src = src[None] + jnp.array([0, N_NODES], jnp.int32)[:, None, None, None]
  dst = _chunk_indices(edge_index[1].astype(jnp.int32), JUNK_ROW)
  dst = jnp.broadcast_to(dst[None], (NUM_CORES,) + dst.shape)

  xs = jnp.concatenate([x[:, :HALF], x[:, HALF:]], axis=0)
  p = _sc_aggregate(src, dst, xs)
  h1 = _mlp1(p, W1, b1.reshape(1, D_FEAT))
  p2 = _sc_aggregate(src, dst, h1.reshape(NUM_CORES * N_NODES, HALF))
  return _mlp2(p2, W2, b2.reshape(1, D_FEAT), W3, b3.reshape(1, N_CLASS))


# speedup vs baseline: 1.5321x; 1.0065x over previous
import functools

import jax
import jax.numpy as jnp
from jax import lax
from jax.experimental import pallas as pl
from jax.experimental.pallas import tpu as pltpu
from jax.experimental.pallas import tpu_sc as plsc

N_NODES = 10000
N_EDGES = 320000
D_FEAT = 128
HALF = D_FEAT // 2
N_CLASS = 40

NUM_CORES = 2
NUM_SUBCORES = 16
EDGES_PER_TILE = N_EDGES // NUM_SUBCORES
CHUNK = 128
NCHUNK = 160
PAD_EDGES = NCHUNK * CHUNK
NBUF = 8
GROUP = 16
NGROUP = NCHUNK // GROUP
ACC_ROWS = 10016
JUNK_ROW = N_NODES
ROWS_MAIN = 632
ROWS_TAIL = N_NODES - 15 * ROWS_MAIN


def _sc_aggregate_body(src_hbm, dst_hbm, feat_hbm, out_hbm,
                       sidx_v, didx_v, rows_v, acc_s, *sems):
  cid = lax.axis_index("c")
  sid = lax.axis_index("s")

  def _refill(q):
    slot = lax.rem(q, 2)
    pltpu.sync_copy(src_hbm.at[cid, sid, pl.ds(q * GROUP, GROUP)],
                    sidx_v.at[slot])
    pltpu.sync_copy(dst_hbm.at[cid, sid, pl.ds(q * GROUP, GROUP)],
                    didx_v.at[slot])

  _refill(0)
  _refill(1)

  row0 = sid * ROWS_MAIN
  feat0 = cid * N_NODES + row0

  def _init(nrows):
    pltpu.sync_copy(feat_hbm.at[pl.ds(feat0, nrows)],
                    acc_s.at[pl.ds(row0, nrows)])

  @pl.when(sid < NUM_SUBCORES - 1)
  def _():
    _init(ROWS_MAIN)

  @pl.when(sid == NUM_SUBCORES - 1)
  def _():
    _init(ROWS_TAIL)

  plsc.subcore_barrier()

  def _sidx(c):
    return sidx_v.at[lax.rem(c // GROUP, 2), lax.rem(c, GROUP)]

  def _didx(c):
    return didx_v.at[lax.rem(c // GROUP, 2), lax.rem(c, GROUP)]

  def _gather(c, b):
    pltpu.async_copy(feat_hbm.at[_sidx(c)], rows_v.at[b], sems[b])

  def _wait_gather(c, b):
    pltpu.make_async_copy(feat_hbm.at[_sidx(c)], rows_v.at[b],
                          sems[b]).wait()

  def _scatter(c, b):
    pltpu.async_copy(rows_v.at[b], acc_s.at[_didx(c)], sems[NBUF + b],
                     add=True)

  def _wait_scatter(c, b):
    pltpu.make_async_copy(rows_v.at[b], acc_s.at[_didx(c)],
                          sems[NBUF + b]).wait()

  for b in range(NBUF - 2):
    _gather(b, b)

  def body(i, carry):
    c0 = NBUF * i
    for b in range(NBUF):
      c = c0 + b
      _wait_gather(c, b)
      _scatter(c, b)
      pb = (b - 2) % NBUF
      cprev = c - 2

      @pl.when(cprev >= 0)
      def _():
        _wait_scatter(cprev, pb)

      @pl.when(c + NBUF - 2 < NCHUNK)
      def _():
        _gather(c + NBUF - 2, pb)

      q = c // GROUP + 1

      @pl.when((lax.rem(c, GROUP) == 4) & (q >= 2) & (q < NGROUP))
      def _():
        _refill(q)

    return carry

  lax.fori_loop(0, NCHUNK // NBUF, body, 0, unroll=False)

  _wait_scatter(NCHUNK - 2, (NCHUNK - 2) % NBUF)
  _wait_scatter(NCHUNK - 1, (NCHUNK - 1) % NBUF)

  plsc.subcore_barrier()

  @pl.when(sid < NUM_SUBCORES - 1)
  def _():
    pltpu.sync_copy(acc_s.at[pl.ds(row0, ROWS_MAIN)],
                    out_hbm.at[cid, pl.ds(row0, ROWS_MAIN)])

  @pl.when(sid == NUM_SUBCORES - 1)
  def _():
    pltpu.sync_copy(acc_s.at[pl.ds(row0, ROWS_TAIL)],
                    out_hbm.at[cid, pl.ds(row0, ROWS_TAIL)])


_sc_aggregate = functools.partial(
    pl.kernel,
    out_type=jax.ShapeDtypeStruct((NUM_CORES, N_NODES, HALF), jnp.float32),
    mesh=plsc.VectorSubcoreMesh(core_axis_name="c", subcore_axis_name="s"),
    compiler_params=pltpu.CompilerParams(use_tc_tiling_on_sc=False),
    scratch_types=[
        pltpu.VMEM((2, GROUP, CHUNK), jnp.int32),
        pltpu.VMEM((2, GROUP, CHUNK), jnp.int32),
        pltpu.VMEM((NBUF, CHUNK, HALF), jnp.float32),
        pltpu.VMEM_SHARED((ACC_ROWS, HALF), jnp.float32),
    ] + [pltpu.SemaphoreType.DMA] * (2 * NBUF),
)(_sc_aggregate_body)


ROW_BLK = 2000


def _mlp1_body(p_ref, w_ref, b_ref, out_ref):
  a = jnp.concatenate([p_ref[0], p_ref[1]], axis=1)
  h = jnp.dot(a, w_ref[...], preferred_element_type=jnp.float32) + b_ref[...]
  h = jnp.maximum(h, 0.0)
  out_ref[0, :, :] = h[:, :HALF]
  out_ref[1, :, :] = h[:, HALF:]


def _mlp2_body(p_ref, w2_ref, b2_ref, w3_ref, b3_ref, out_ref):
  a = jnp.concatenate([p_ref[0], p_ref[1]], axis=1)
  h2 = jnp.dot(a, w2_ref[...], preferred_element_type=jnp.float32)
  h2 = jnp.maximum(h2 + b2_ref[...], 0.0)
  logits = jnp.dot(h2, w3_ref[...], preferred_element_type=jnp.float32)
  logits = logits + b3_ref[...]
  m = jnp.max(logits, axis=1, keepdims=True)
  lse = m + jnp.log(jnp.sum(jnp.exp(logits - m), axis=1, keepdims=True))
  out_ref[...] = logits - lse


def _split_block(i):
  return (0, i, 0)


def _row_block(i):
  return (i, 0)


def _full_block(i):
  return (0, 0)


_mlp1 = pl.pallas_call(
    _mlp1_body,
    grid=(N_NODES // ROW_BLK,),
    in_specs=[
        pl.BlockSpec((NUM_CORES, ROW_BLK, HALF), _split_block),
        pl.BlockSpec((D_FEAT, D_FEAT), _full_block),
        pl.BlockSpec((1, D_FEAT), _full_block),
    ],
    out_specs=pl.BlockSpec((NUM_CORES, ROW_BLK, HALF), _split_block),
    out_shape=jax.ShapeDtypeStruct((NUM_CORES, N_NODES, HALF), jnp.float32),
)

_mlp2 = pl.pallas_call(
    _mlp2_body,
    grid=(N_NODES // ROW_BLK,),
    in_specs=[
        pl.BlockSpec((NUM_CORES, ROW_BLK, HALF), _split_block),
        pl.BlockSpec((D_FEAT, D_FEAT), _full_block),
        pl.BlockSpec((1, D_FEAT), _full_block),
        pl.BlockSpec((D_FEAT, N_CLASS), _full_block),
        pl.BlockSpec((1, N_CLASS), _full_block),
    ],
    out_specs=pl.BlockSpec((ROW_BLK, N_CLASS), _row_block),
    out_shape=jax.ShapeDtypeStruct((N_NODES, N_CLASS), jnp.float32),
)


def _chunk_indices(idx, pad_value):
  per_tile = idx.reshape(NUM_SUBCORES, EDGES_PER_TILE)
  padded = jnp.pad(per_tile, ((0, 0), (0, PAD_EDGES - EDGES_PER_TILE)),
                   constant_values=pad_value)
  return padded.reshape(NUM_SUBCORES, NCHUNK, CHUNK)


@jax.jit
def kernel(x, edge_index, W1, b1, W2, b2, W3, b3):
  src = _chunk_indices(edge_index[0].astype(jnp.int32), 0)
  src = src[None] + jnp.array([0, N_NODES], jnp.int32)[:, None, None, None]
  dst = _chunk_indices(edge_index[1].astype(jnp.int32), JUNK_ROW)
  dst = jnp.broadcast_to(dst[None], (NUM_CORES,) + dst.shape)

  xs = jnp.concatenate([x[:, :HALF], x[:, HALF:]], axis=0)
  p = _sc_aggregate(src, dst, xs)
  h1 = _mlp1(p, W1, b1.reshape(1, D_FEAT))
  p2 = _sc_aggregate(src, dst, h1.reshape(NUM_CORES * N_NODES, HALF))
  return _mlp2(p2, W2, b2.reshape(1, D_FEAT), W3, b3.reshape(1, N_CLASS))


# speedup vs baseline: 2.3627x; 1.5422x over previous
import functools

import jax
import jax.numpy as jnp
from jax import lax
from jax.experimental import pallas as pl
from jax.experimental.pallas import tpu as pltpu
from jax.experimental.pallas import tpu_sc as plsc

N_NODES = 10000
N_EDGES = 320000
D_FEAT = 128
HALF = D_FEAT // 2
N_CLASS = 40

NUM_CORES = 2
NUM_SUBCORES = 16
EDGES_PER_TILE = N_EDGES // NUM_SUBCORES
CHUNK = 128
NCHUNK = 157
NCHUNK_PAD = 160
PAD_EDGES = NCHUNK_PAD * CHUNK
NBUF = 8
GROUP = 16
NGROUP = -(-NCHUNK // GROUP)
ACC_ROWS = 10016
JUNK_ROW = N_NODES
ROWS_MAIN = 632
ROWS_TAIL = N_NODES - 15 * ROWS_MAIN

NMAIN = 152


def _sc_aggregate_body(src_hbm, dst_hbm, feat_hbm, zeros_hbm, out_hbm,
                       sidx_v, didx_v, rows_v, acc_s, *sems):
  cid = lax.axis_index("c")
  sid = lax.axis_index("s")

  def _refill(q):
    slot = lax.rem(q, 2)
    pltpu.sync_copy(src_hbm.at[cid, sid, pl.ds(q * GROUP, GROUP)],
                    sidx_v.at[slot])
    pltpu.sync_copy(dst_hbm.at[sid, pl.ds(q * GROUP, GROUP)],
                    didx_v.at[slot])

  _refill(0)
  _refill(1)

  row0 = sid * ROWS_MAIN

  @pl.when(sid < NUM_SUBCORES - 1)
  def _():
    pltpu.sync_copy(zeros_hbm.at[pl.ds(row0, ROWS_MAIN)],
                    acc_s.at[pl.ds(row0, ROWS_MAIN)])

  @pl.when(sid == NUM_SUBCORES - 1)
  def _():
    pltpu.sync_copy(zeros_hbm.at[pl.ds(row0, ROWS_TAIL)],
                    acc_s.at[pl.ds(row0, ROWS_TAIL)])

  plsc.subcore_barrier()

  def _sidx(c):
    return sidx_v.at[lax.rem(c // GROUP, 2), lax.rem(c, GROUP)]

  def _didx(c):
    return didx_v.at[lax.rem(c // GROUP, 2), lax.rem(c, GROUP)]

  def _gather(c, b):
    pltpu.async_copy(feat_hbm.at[_sidx(c)], rows_v.at[b], sems[b])

  def _wait_gather(c, b):
    pltpu.make_async_copy(feat_hbm.at[_sidx(c)], rows_v.at[b],
                          sems[b]).wait()

  def _scatter(c, b):
    pltpu.async_copy(rows_v.at[b], acc_s.at[_didx(c)], sems[NBUF + b],
                     add=True)

  def _wait_scatter(c, b):
    pltpu.make_async_copy(rows_v.at[b], acc_s.at[_didx(c)],
                          sems[NBUF + b]).wait()

  for b in range(NBUF - 2):
    _gather(b, b)

  def body(i, carry):
    c0 = NBUF * i
    for b in range(NBUF):
      c = c0 + b
      _wait_gather(c, b)
      _scatter(c, b)
      pb = (b - 2) % NBUF
      cprev = c - 2

      @pl.when(cprev >= 0)
      def _():
        _wait_scatter(cprev, pb)

      @pl.when(c + NBUF - 2 < NCHUNK)
      def _():
        _gather(c + NBUF - 2, pb)

      q = c // GROUP + 1

      @pl.when((lax.rem(c, GROUP) == 4) & (q >= 2) & (q < NGROUP))
      def _():
        _refill(q)

    return carry

  lax.fori_loop(0, NMAIN // NBUF, body, 0, unroll=False)

  for c in range(NMAIN, NCHUNK):
    b = c % NBUF
    _wait_gather(c, b)
    _scatter(c, b)
    _wait_scatter(c - 2, (c - 2) % NBUF)

  _wait_scatter(NCHUNK - 2, (NCHUNK - 2) % NBUF)
  _wait_scatter(NCHUNK - 1, (NCHUNK - 1) % NBUF)

  plsc.subcore_barrier()

  @pl.when(sid < NUM_SUBCORES - 1)
  def _():
    pltpu.sync_copy(acc_s.at[pl.ds(row0, ROWS_MAIN)],
                    out_hbm.at[cid, pl.ds(row0, ROWS_MAIN)])

  @pl.when(sid == NUM_SUBCORES - 1)
  def _():
    pltpu.sync_copy(acc_s.at[pl.ds(row0, ROWS_TAIL)],
                    out_hbm.at[cid, pl.ds(row0, ROWS_TAIL)])


_sc_aggregate = functools.partial(
    pl.kernel,
    out_type=jax.ShapeDtypeStruct((NUM_CORES, N_NODES, HALF), jnp.float32),
    mesh=plsc.VectorSubcoreMesh(core_axis_name="c", subcore_axis_name="s"),
    compiler_params=pltpu.CompilerParams(use_tc_tiling_on_sc=False),
    scratch_types=[
        pltpu.VMEM((2, GROUP, CHUNK), jnp.int32),
        pltpu.VMEM((2, GROUP, CHUNK), jnp.int32),
        pltpu.VMEM((NBUF, CHUNK, HALF), jnp.float32),
        pltpu.VMEM_SHARED((ACC_ROWS, HALF), jnp.float32),
    ] + [pltpu.SemaphoreType.DMA] * (2 * NBUF),
)(_sc_aggregate_body)


ROW_BLK = 2000


def _mlp1_body(p_ref, x_ref, w_ref, b_ref, out_ref):
  a = jnp.concatenate([p_ref[0], p_ref[1]], axis=1) + x_ref[...]
  h = jnp.dot(a, w_ref[...], preferred_element_type=jnp.float32) + b_ref[...]
  out_ref[...] = jnp.maximum(h, 0.0)


def _mlp2_body(p_ref, h1_ref, w2_ref, b2_ref, w3_ref, b3_ref, out_ref):
  a = jnp.concatenate([p_ref[0], p_ref[1]], axis=1) + h1_ref[...]
  h2 = jnp.dot(a, w2_ref[...], preferred_element_type=jnp.float32)
  h2 = jnp.maximum(h2 + b2_ref[...], 0.0)
  logits = jnp.dot(h2, w3_ref[...], preferred_element_type=jnp.float32)
  logits = logits + b3_ref[...]
  m = jnp.max(logits, axis=1, keepdims=True)
  lse = m + jnp.log(jnp.sum(jnp.exp(logits - m), axis=1, keepdims=True))
  out_ref[...] = logits - lse


def _split_block(i):
  return (0, i, 0)


def _row_block(i):
  return (i, 0)


def _full_block(i):
  return (0, 0)


_mlp1 = pl.pallas_call(
    _mlp1_body,
    grid=(N_NODES // ROW_BLK,),
    in_specs=[
        pl.BlockSpec((NUM_CORES, ROW_BLK, HALF), _split_block),
        pl.BlockSpec((ROW_BLK, D_FEAT), _row_block),
        pl.BlockSpec((D_FEAT, D_FEAT), _full_block),
        pl.BlockSpec((1, D_FEAT), _full_block),
    ],
    out_specs=pl.BlockSpec((ROW_BLK, D_FEAT), _row_block),
    out_shape=jax.ShapeDtypeStruct((N_NODES, D_FEAT), jnp.float32),
)

_mlp2 = pl.pallas_call(
    _mlp2_body,
    grid=(N_NODES // ROW_BLK,),
    in_specs=[
        pl.BlockSpec((NUM_CORES, ROW_BLK, HALF), _split_block),
        pl.BlockSpec((ROW_BLK, D_FEAT), _row_block),
        pl.BlockSpec((D_FEAT, D_FEAT), _full_block),
        pl.BlockSpec((1, D_FEAT), _full_block),
        pl.BlockSpec((D_FEAT, N_CLASS), _full_block),
        pl.BlockSpec((1, N_CLASS), _full_block),
    ],
    out_specs=pl.BlockSpec((ROW_BLK, N_CLASS), _row_block),
    out_shape=jax.ShapeDtypeStruct((N_NODES, N_CLASS), jnp.float32),
)


def _chunk_indices(idx, pad_value):
  per_tile = idx.reshape(NUM_SUBCORES, EDGES_PER_TILE)
  padded = jnp.pad(per_tile, ((0, 0), (0, PAD_EDGES - EDGES_PER_TILE)),
                   constant_values=pad_value)
  return padded.reshape(NUM_SUBCORES, NCHUNK_PAD, CHUNK)


@jax.jit
def kernel(x, edge_index, W1, b1, W2, b2, W3, b3):
  src = _chunk_indices(edge_index[0].astype(jnp.int32), 0)
  src = 2 * src[None] + jnp.arange(NUM_CORES, dtype=jnp.int32)[:, None,
                                                               None, None]
  dst = _chunk_indices(edge_index[1].astype(jnp.int32), JUNK_ROW)
  zeros = jnp.zeros((N_NODES, HALF), jnp.float32)

  p = _sc_aggregate(src, dst, x.reshape(NUM_CORES * N_NODES, HALF), zeros)
  h1 = _mlp1(p, x, W1, b1.reshape(1, D_FEAT))
  p2 = _sc_aggregate(src, dst, h1.reshape(NUM_CORES * N_NODES, HALF), zeros)
  return _mlp2(p2, h1, W2, b2.reshape(1, D_FEAT), W3, b3.reshape(1, N_CLASS))


# speedup vs baseline: 2.4068x; 1.0186x over previous
import functools

import jax
import jax.numpy as jnp
from jax import lax
from jax.experimental import pallas as pl
from jax.experimental.pallas import tpu as pltpu
from jax.experimental.pallas import tpu_sc as plsc

N_NODES = 10000
N_EDGES = 320000
D_FEAT = 128
HALF = D_FEAT // 2
N_CLASS = 40

NUM_CORES = 2
NUM_SUBCORES = 16
EDGES_PER_TILE = N_EDGES // NUM_SUBCORES
CHUNK = 128
NCHUNK = 157
NCHUNK_PAD = 160
PAD_EDGES = NCHUNK_PAD * CHUNK
NBUF = 8
GROUP = 32
NGROUP = -(-NCHUNK // GROUP)
ACC_ROWS = 10016
JUNK_ROW = N_NODES
ROWS_MAIN = 632
ROWS_TAIL = N_NODES - 15 * ROWS_MAIN

NMAIN = 152


def _sc_aggregate_body(src_hbm, dst_hbm, feat_hbm, zeros_hbm, out_hbm,
                       sidx_v, didx_v, rows_v, acc_s, *sems):
  cid = lax.axis_index("c")
  sid = lax.axis_index("s")

  def _refill(q):
    slot = lax.rem(q, 2)
    pltpu.sync_copy(src_hbm.at[cid, sid, pl.ds(q * GROUP, GROUP)],
                    sidx_v.at[slot])
    pltpu.sync_copy(dst_hbm.at[sid, pl.ds(q * GROUP, GROUP)],
                    didx_v.at[slot])

  _refill(0)
  _refill(1)

  row0 = sid * ROWS_MAIN

  @pl.when(sid < NUM_SUBCORES - 1)
  def _():
    pltpu.sync_copy(zeros_hbm.at[pl.ds(row0, ROWS_MAIN)],
                    acc_s.at[pl.ds(row0, ROWS_MAIN)])

  @pl.when(sid == NUM_SUBCORES - 1)
  def _():
    pltpu.sync_copy(zeros_hbm.at[pl.ds(row0, ROWS_TAIL)],
                    acc_s.at[pl.ds(row0, ROWS_TAIL)])

  plsc.subcore_barrier()

  def _sidx(c):
    return sidx_v.at[lax.rem(c // GROUP, 2), lax.rem(c, GROUP)]

  def _didx(c):
    return didx_v.at[lax.rem(c // GROUP, 2), lax.rem(c, GROUP)]

  def _gather(c, b):
    pltpu.async_copy(feat_hbm.at[_sidx(c)], rows_v.at[b], sems[b])

  def _wait_gather(c, b):
    pltpu.make_async_copy(feat_hbm.at[_sidx(c)], rows_v.at[b],
                          sems[b]).wait()

  def _scatter(c, b):
    pltpu.async_copy(rows_v.at[b], acc_s.at[_didx(c)], sems[NBUF + b],
                     add=True)

  def _wait_scatter(c, b):
    pltpu.make_async_copy(rows_v.at[b], acc_s.at[_didx(c)],
                          sems[NBUF + b]).wait()

  for b in range(NBUF - 2):
    _gather(b, b)

  def body(i, carry):
    c0 = NBUF * i
    for b in range(NBUF):
      c = c0 + b
      _wait_gather(c, b)
      _scatter(c, b)
      pb = (b - 2) % NBUF
      cprev = c - 2

      @pl.when(cprev >= 0)
      def _():
        _wait_scatter(cprev, pb)

      @pl.when(c + NBUF - 2 < NCHUNK)
      def _():
        _gather(c + NBUF - 2, pb)

      q = c // GROUP + 1

      @pl.when((lax.rem(c, GROUP) == 4) & (q >= 2) & (q < NGROUP))
      def _():
        _refill(q)

    return carry

  lax.fori_loop(0, NMAIN // NBUF, body, 0, unroll=False)

  for c in range(NMAIN, NCHUNK):
    b = c % NBUF
    _wait_gather(c, b)
    _scatter(c, b)
    _wait_scatter(c - 2, (c - 2) % NBUF)

  _wait_scatter(NCHUNK - 2, (NCHUNK - 2) % NBUF)
  _wait_scatter(NCHUNK - 1, (NCHUNK - 1) % NBUF)

  plsc.subcore_barrier()

  @pl.when(sid < NUM_SUBCORES - 1)
  def _():
    pltpu.sync_copy(acc_s.at[pl.ds(row0, ROWS_MAIN)],
                    out_hbm.at[cid, pl.ds(row0, ROWS_MAIN)])

  @pl.when(sid == NUM_SUBCORES - 1)
  def _():
    pltpu.sync_copy(acc_s.at[pl.ds(row0, ROWS_TAIL)],
                    out_hbm.at[cid, pl.ds(row0, ROWS_TAIL)])


_sc_aggregate = functools.partial(
    pl.kernel,
    out_type=jax.ShapeDtypeStruct((NUM_CORES, N_NODES, HALF), jnp.float32),
    mesh=plsc.VectorSubcoreMesh(core_axis_name="c", subcore_axis_name="s"),
    compiler_params=pltpu.CompilerParams(use_tc_tiling_on_sc=False),
    scratch_types=[
        pltpu.VMEM((2, GROUP, CHUNK), jnp.int32),
        pltpu.VMEM((2, GROUP, CHUNK), jnp.int32),
        pltpu.VMEM((NBUF, CHUNK, HALF), jnp.float32),
        pltpu.VMEM_SHARED((ACC_ROWS, HALF), jnp.float32),
    ] + [pltpu.SemaphoreType.DMA] * (2 * NBUF),
)(_sc_aggregate_body)


ROW_BLK = 2000


def _mlp1_body(p_ref, x_ref, w_ref, b_ref, out_ref):
  a = jnp.concatenate([p_ref[0], p_ref[1]], axis=1) + x_ref[...]
  h = jnp.dot(a, w_ref[...], preferred_element_type=jnp.float32) + b_ref[...]
  out_ref[...] = jnp.maximum(h, 0.0)


def _mlp2_body(p_ref, h1_ref, w2_ref, b2_ref, w3_ref, b3_ref, out_ref):
  a = jnp.concatenate([p_ref[0], p_ref[1]], axis=1) + h1_ref[...]
  h2 = jnp.dot(a, w2_ref[...], preferred_element_type=jnp.float32)
  h2 = jnp.maximum(h2 + b2_ref[...], 0.0)
  logits = jnp.dot(h2, w3_ref[...], preferred_element_type=jnp.float32)
  logits = logits + b3_ref[...]
  m = jnp.max(logits, axis=1, keepdims=True)
  lse = m + jnp.log(jnp.sum(jnp.exp(logits - m), axis=1, keepdims=True))
  out_ref[...] = logits - lse


def _split_block(i):
  return (0, i, 0)


def _row_block(i):
  return (i, 0)


def _full_block(i):
  return (0, 0)


_mlp1 = pl.pallas_call(
    _mlp1_body,
    grid=(N_NODES // ROW_BLK,),
    in_specs=[
        pl.BlockSpec((NUM_CORES, ROW_BLK, HALF), _split_block),
        pl.BlockSpec((ROW_BLK, D_FEAT), _row_block),
        pl.BlockSpec((D_FEAT, D_FEAT), _full_block),
        pl.BlockSpec((1, D_FEAT), _full_block),
    ],
    out_specs=pl.BlockSpec((ROW_BLK, D_FEAT), _row_block),
    out_shape=jax.ShapeDtypeStruct((N_NODES, D_FEAT), jnp.float32),
)

_mlp2 = pl.pallas_call(
    _mlp2_body,
    grid=(N_NODES // ROW_BLK,),
    in_specs=[
        pl.BlockSpec((NUM_CORES, ROW_BLK, HALF), _split_block),
        pl.BlockSpec((ROW_BLK, D_FEAT), _row_block),
        pl.BlockSpec((D_FEAT, D_FEAT), _full_block),
        pl.BlockSpec((1, D_FEAT), _full_block),
        pl.BlockSpec((D_FEAT, N_CLASS), _full_block),
        pl.BlockSpec((1, N_CLASS), _full_block),
    ],
    out_specs=pl.BlockSpec((ROW_BLK, N_CLASS), _row_block),
    out_shape=jax.ShapeDtypeStruct((N_NODES, N_CLASS), jnp.float32),
)


def _chunk_indices(idx, pad_value):
  per_tile = idx.reshape(NUM_SUBCORES, EDGES_PER_TILE)
  padded = jnp.pad(per_tile, ((0, 0), (0, PAD_EDGES - EDGES_PER_TILE)),
                   constant_values=pad_value)
  return padded.reshape(NUM_SUBCORES, NCHUNK_PAD, CHUNK)


@jax.jit
def kernel(x, edge_index, W1, b1, W2, b2, W3, b3):
  src = _chunk_indices(edge_index[0].astype(jnp.int32), 0)
  src = 2 * src[None] + jnp.arange(NUM_CORES, dtype=jnp.int32)[:, None,
                                                               None, None]
  dst = _chunk_indices(edge_index[1].astype(jnp.int32), JUNK_ROW)
  zeros = jnp.zeros((N_NODES, HALF), jnp.float32)

  p = _sc_aggregate(src, dst, x.reshape(NUM_CORES * N_NODES, HALF), zeros)
  h1 = _mlp1(p, x, W1, b1.reshape(1, D_FEAT))
  p2 = _sc_aggregate(src, dst, h1.reshape(NUM_CORES * N_NODES, HALF), zeros)
  return _mlp2(p2, h1, W2, b2.reshape(1, D_FEAT), W3, b3.reshape(1, N_CLASS))
